# baseline (speedup 1.0000x reference)
import functools

import jax
import jax.numpy as jnp
from jax import lax
from jax.experimental import pallas as pl
from jax.experimental.pallas import tpu as pltpu
from jax.experimental.pallas import tpu_sc as plsc

_BATCH = 16384
_HIST = 50
_EMB = 32

_NC = 2
_NS = 16
_NW = _NC * _NS
_BPW = _BATCH // _NW
_K = 128
_NK = _BPW // _K

_mesh = plsc.VectorSubcoreMesh(core_axis_name="c", subcore_axis_name="s")

_V = 1000000
_CH = 256
_NCH = _V // _CH
_REM = _V - _NCH * _CH


@functools.partial(
    pl.kernel,
    mesh=_mesh,
    out_type=jax.ShapeDtypeStruct((_V // 4, 128), jnp.float32),
    compiler_params=pltpu.CompilerParams(
        use_tc_tiling_on_sc=True, needs_layout_passes=False),
    scratch_types=[
        pltpu.VMEM((2, _CH, _EMB), jnp.float32),
        pltpu.VMEM((2, _CH // 4, 128), jnp.float32),
        pltpu.SemaphoreType.DMA,
        pltpu.SemaphoreType.DMA,
    ],
)
def _sc_repack(t_hbm, out_hbm, v1, v2, isem, osem):
    wid = lax.axis_index("s") * _NC + lax.axis_index("c")

    def in_copy(c, s):
        return pltpu.make_async_copy(
            t_hbm.at[pl.ds(c * _CH, _CH)], v1.at[s], isem)

    def out_copy(c, s):
        return pltpu.make_async_copy(
            v2.at[s], out_hbm.at[pl.ds(c * (_CH // 4), _CH // 4)], osem)

    def shuffle(s):
        @plsc.parallel_loop(0, _CH, unroll=4)
        def qbody(q):
            p = q // 4
            g = q % 4
            v2[s, p, pl.ds(g * 32, 16)] = v1[s, q, pl.ds(0, 16)]
            v2[s, p, pl.ds(g * 32 + 16, 16)] = v1[s, q, pl.ds(16, 16)]

    n = 122 + jnp.where(wid < _NCH - 122 * _NW, 1, 0)
    in_copy(wid, 0).start()
    in_copy(wid + _NW, 1).start()

    def body(t, carry):
        c = wid + t * _NW
        s = t % 2

        @pl.when(c < _NCH)
        def _():
            in_copy(c, s).wait()

            @pl.when(t >= 2)
            def _():
                out_copy(c - 2 * _NW, s).wait()
            shuffle(s)

            @pl.when(c + 2 * _NW < _NCH)
            def _():
                in_copy(c + 2 * _NW, s).start()
            out_copy(c, s).start()
        return carry

    lax.fori_loop(0, 123, body, 0)
    out_copy(wid + (n - 2) * _NW, (n - 2) % 2).wait()
    out_copy(wid + (n - 1) * _NW, (n - 1) % 2).wait()

    @pl.when(wid == 2)
    def _():
        in_copy_t = pltpu.make_async_copy(
            t_hbm.at[pl.ds(_NCH * _CH, _REM)],
            v1.at[0, pl.ds(0, _REM)], isem)
        in_copy_t.start()
        in_copy_t.wait()

        @plsc.parallel_loop(0, _REM, unroll=4)
        def qbody(q):
            p = q // 4
            g = q % 4
            v2[0, p, pl.ds(g * 32, 16)] = v1[0, q, pl.ds(0, 16)]
            v2[0, p, pl.ds(g * 32 + 16, 16)] = v1[0, q, pl.ds(16, 16)]

        out_copy_t = pltpu.make_async_copy(
            v2.at[0, pl.ds(0, _REM // 4)],
            out_hbm.at[pl.ds(_NCH * (_CH // 4), _REM // 4)], osem)
        out_copy_t.start()
        out_copy_t.wait()


@functools.partial(
    pl.kernel,
    mesh=_mesh,
    out_type=jax.ShapeDtypeStruct((_HIST, _EMB, _BATCH), jnp.float32),
    compiler_params=pltpu.CompilerParams(
        use_tc_tiling_on_sc=False, needs_layout_passes=False),
    scratch_types=[
        pltpu.VMEM((_NK, _HIST, _K), jnp.int32),
        pltpu.VMEM((2, _BPW, _EMB), jnp.float32),
        pltpu.VMEM((2, _EMB, _BPW), jnp.float32),
        pltpu.SemaphoreType.DMA,
        pltpu.SemaphoreType.DMA,
        pltpu.SemaphoreType.DMA,
    ],
)
def _sc_gather(xt_hbm, table_hbm, out_hbm, idx_v, rows_v, tbuf_v, gsem, isem, osem):
    wid = lax.axis_index("s") * _NC + lax.axis_index("c")
    b0 = wid * _BPW

    for k in range(_NK):
        pltpu.async_copy(
            xt_hbm.at[:, pl.ds(b0 + k * _K, _K)], idx_v.at[k], isem)
    for k in range(_NK):
        pltpu.make_async_copy(
            xt_hbm.at[:, pl.ds(b0 + k * _K, _K)], idx_v.at[k], isem).wait()

    iota = lax.iota(jnp.int32, 16)

    def fire_g(h, s):
        for k in range(_NK):
            pltpu.async_copy(
                table_hbm.at[idx_v.at[k, h]],
                rows_v.at[s, pl.ds(k * _K, _K)], gsem)

    def wait_g(h, s):
        for k in range(_NK):
            pltpu.make_async_copy(
                table_hbm.at[idx_v.at[k, h]],
                rows_v.at[s, pl.ds(k * _K, _K)], gsem).wait()

    def fire_wb(h, s):
        pltpu.async_copy(tbuf_v.at[s], out_hbm.at[h, :, pl.ds(b0, _BPW)], osem)

    def wait_wb(h, s):
        pltpu.make_async_copy(
            tbuf_v.at[s], out_hbm.at[h, :, pl.ds(b0, _BPW)], osem).wait()

    def transpose(s):
        @plsc.parallel_loop(0, _BPW // 16, unroll=4)
        def vbody(v):
            row = v * 16 + iota
            for c in range(_EMB):
                col = (c + iota) & (_EMB - 1)
                vec = plsc.load_gather(rows_v.at[s], [row, col])
                plsc.store_scatter(tbuf_v.at[s], [col, row], vec)

    fire_g(0, 0)
    fire_g(1, 1)
    for h in (0, 1):
        wait_g(h, h)
        transpose(h)
        fire_g(h + 2, h)
        fire_wb(h, h)

    def body(t, carry):
        for s in range(2):
            h = 2 * t + s
            wait_g(h, s)
            wait_wb(h - 2, s)
            transpose(s)
            fire_g(h + 2, s)
            fire_wb(h, s)
        return carry

    lax.fori_loop(1, _HIST // 2 - 1, body, 0)

    for s in range(2):
        h = _HIST - 2 + s
        wait_g(h, s)
        wait_wb(h - 2, s)
        transpose(s)
        fire_wb(h, s)
    wait_wb(_HIST - 2, 0)
    wait_wb(_HIST - 1, 1)


def _mask_body(xt_ref, o_ref):
    o_ref[...] = xt_ref[...] > 0


_mask_call = pl.pallas_call(
    _mask_body,
    out_shape=jax.ShapeDtypeStruct((_HIST, _BATCH), jnp.bool_),
    grid=(8,),
    in_specs=[pl.BlockSpec((_HIST, _BATCH // 8), lambda i: (0, i))],
    out_specs=pl.BlockSpec((_HIST, _BATCH // 8), lambda i: (0, i)),
)


def kernel(x, table):
    xt = x.T.astype(jnp.int32)
    tbl = _sc_repack(table).reshape(_V, _EMB)
    out = _sc_gather(xt, tbl)
    embs = out.transpose(2, 0, 1)
    mask = _mask_call(xt).T
    return embs, mask


# speedup vs baseline: 1.0214x; 1.0214x over previous
import functools

import jax
import jax.numpy as jnp
from jax import lax
from jax.experimental import pallas as pl
from jax.experimental.pallas import tpu as pltpu
from jax.experimental.pallas import tpu_sc as plsc

_BATCH = 16384
_HIST = 50
_EMB = 32

_NC = 2
_NS = 16
_NW = _NC * _NS
_BPW = _BATCH // _NW
_K = 128
_NK = _BPW // _K

_mesh = plsc.VectorSubcoreMesh(core_axis_name="c", subcore_axis_name="s")


@functools.partial(
    pl.kernel,
    mesh=_mesh,
    out_type=jax.ShapeDtypeStruct((_HIST, _EMB, _BATCH), jnp.float32),
    compiler_params=pltpu.CompilerParams(
        use_tc_tiling_on_sc=False, needs_layout_passes=False),
    scratch_types=[
        pltpu.VMEM((_NK, _HIST, _K), jnp.int32),
        pltpu.VMEM((2, _BPW, _EMB), jnp.float32),
        pltpu.VMEM((2, _EMB, _BPW), jnp.float32),
        pltpu.SemaphoreType.DMA,
        pltpu.SemaphoreType.DMA,
        pltpu.SemaphoreType.DMA,
    ],
)
def _sc_gather(xt_hbm, table_hbm, out_hbm, idx_v, rows_v, tbuf_v, gsem, isem, osem):
    wid = lax.axis_index("s") * _NC + lax.axis_index("c")
    b0 = wid * _BPW

    for k in range(_NK):
        pltpu.async_copy(
            xt_hbm.at[:, pl.ds(b0 + k * _K, _K)], idx_v.at[k], isem)
    for k in range(_NK):
        pltpu.make_async_copy(
            xt_hbm.at[:, pl.ds(b0 + k * _K, _K)], idx_v.at[k], isem).wait()

    iota = lax.iota(jnp.int32, 16)

    def fire_g(h, s):
        for k in range(_NK):
            pltpu.async_copy(
                table_hbm.at[idx_v.at[k, h]],
                rows_v.at[s, pl.ds(k * _K, _K)], gsem)

    def wait_g(h, s):
        for k in range(_NK):
            pltpu.make_async_copy(
                table_hbm.at[idx_v.at[k, h]],
                rows_v.at[s, pl.ds(k * _K, _K)], gsem).wait()

    def fire_wb(h, s):
        pltpu.async_copy(tbuf_v.at[s], out_hbm.at[h, :, pl.ds(b0, _BPW)], osem)

    def wait_wb(h, s):
        pltpu.make_async_copy(
            tbuf_v.at[s], out_hbm.at[h, :, pl.ds(b0, _BPW)], osem).wait()

    def transpose(s):
        @plsc.parallel_loop(0, _BPW // 16, unroll=4)
        def vbody(v):
            row = v * 16 + iota
            for c in range(_EMB):
                col = (c + iota) & (_EMB - 1)
                vec = plsc.load_gather(rows_v.at[s], [row, col])
                plsc.store_scatter(tbuf_v.at[s], [col, row], vec)

    fire_g(0, 0)
    fire_g(1, 1)
    for h in (0, 1):
        wait_g(h, h)
        transpose(h)
        fire_g(h + 2, h)
        fire_wb(h, h)

    def body(t, carry):
        for s in range(2):
            h = 2 * t + s
            wait_g(h, s)
            wait_wb(h - 2, s)
            transpose(s)
            fire_g(h + 2, s)
            fire_wb(h, s)
        return carry

    lax.fori_loop(1, _HIST // 2 - 1, body, 0)

    for s in range(2):
        h = _HIST - 2 + s
        wait_g(h, s)
        wait_wb(h - 2, s)
        transpose(s)
        fire_wb(h, s)
    wait_wb(_HIST - 2, 0)
    wait_wb(_HIST - 1, 1)


def _mask_body(xt_ref, o_ref):
    o_ref[...] = xt_ref[...] > 0


_mask_call = pl.pallas_call(
    _mask_body,
    out_shape=jax.ShapeDtypeStruct((_HIST, _BATCH), jnp.bool_),
    grid=(8,),
    in_specs=[pl.BlockSpec((_HIST, _BATCH // 8), lambda i: (0, i))],
    out_specs=pl.BlockSpec((_HIST, _BATCH // 8), lambda i: (0, i)),
)


def kernel(x, table):
    xt = x.T.astype(jnp.int32)
    out = _sc_gather(xt, table)
    embs = out.transpose(2, 0, 1)
    mask = _mask_call(xt).T
    return embs, mask
